# Initial kernel scaffold; baseline (speedup 1.0000x reference)
import jax
import jax.numpy as jnp
from jax.experimental import pallas as pl


def kernel(context, glove_table, unk_table):
    raise NotImplementedError("write your pallas kernel here")


# speedup vs baseline: 28.6203x; 28.6203x over previous
import functools

import jax
import jax.numpy as jnp
from jax import lax
from jax.experimental import pallas as pl
from jax.experimental.pallas import tpu as pltpu
from jax.experimental.pallas import tpu_sc as plsc

UNK = 1000
D = 64
SUB = 128
C = 1024
NSUB = C // SUB


def _make_gather(L, NW, per_w):
    nch = per_w // C
    mesh = plsc.VectorSubcoreMesh(core_axis_name="c", subcore_axis_name="s")

    @functools.partial(
        pl.kernel,
        mesh=mesh,
        compiler_params=pltpu.CompilerParams(use_tc_tiling_on_sc=False),
        out_type=jax.ShapeDtypeStruct((L, D), jnp.float32),
        scratch_types=[
            pltpu.VMEM((NSUB, SUB), jnp.int32),
            pltpu.VMEM((C, D), jnp.float32),
            pltpu.SemaphoreType.DMA,
        ],
    )
    def body(ids_hbm, table_hbm, out_hbm, idx_v, rows_v, sem):
        wid = lax.axis_index("s") * 2 + lax.axis_index("c")
        base = wid * per_w

        def chunk(g, carry):
            b0 = pl.multiple_of(base + g * C, SUB)
            pltpu.sync_copy(
                ids_hbm.at[pl.ds(pl.multiple_of(b0 // SUB, 8), NSUB)], idx_v
            )
            cps = [
                pltpu.async_copy(
                    table_hbm.at[idx_v.at[j]],
                    rows_v.at[pl.ds(j * SUB, SUB)],
                    sem,
                )
                for j in range(NSUB)
            ]
            for cp in cps:
                cp.wait()
            pltpu.sync_copy(rows_v, out_hbm.at[pl.ds(b0, C)])
            return carry

        lax.fori_loop(0, nch, chunk, 0)

    return body


def kernel(context, glove_table, unk_table):
    b, t = context.shape
    L = b * t
    NW = 32
    per_w = L // NW
    assert per_w % C == 0
    flat = context.reshape(L // SUB, SUB)
    table = jnp.concatenate([unk_table, glove_table], axis=0)
    out = _make_gather(L, NW, per_w)(flat, table)
    return out.reshape(b, t, D)
